# baseline (speedup 1.0000x reference)
import functools

import jax
import jax.numpy as jnp
from jax import lax
from jax.experimental import pallas as pl
from jax.experimental.pallas import tpu as pltpu
from jax.experimental.pallas import tpu_sc as plsc

B, L, D = 16384, 200, 64
N = B * L
NUM_TIMES = 100000
NUM_WD = 7
NC, NS = 2, 16
NW = NC * NS

P = 4
PB = B // P
NP = PB * L
NPW = NP // NW
K = 800
SUBS = (128, 128, 128, 128, 128, 128, 32)
NCHUNK = NPW // K

TB = 2000
TROWS = NUM_TIMES * D // 128


def _augment_body(wt_ref, wwd_ref, out_ref):
    w = pl.program_id(0)
    out_ref[0, :, :] = wt_ref[...] + wwd_ref[w, :][None, :]


def _augment(w_time2, w_wd2):
    return pl.pallas_call(
        _augment_body,
        grid=(NUM_WD, TROWS // TB),
        in_specs=[
            pl.BlockSpec((TB, 128), lambda w, i: (i, 0)),
            pl.BlockSpec((NUM_WD, 128), lambda w, i: (0, 0)),
        ],
        out_specs=pl.BlockSpec((1, TB, 128), lambda w, i: (w, i, 0)),
        out_shape=jax.ShapeDtypeStruct((NUM_WD, TROWS, 128), jnp.float32),
    )(w_time2, w_wd2)


CB = 1600
IROWS = N // 128


def _combine_body(t_ref, w_ref, out_ref):
    out_ref[...] = w_ref[...] * NUM_TIMES + t_ref[...]


def _combine(t2, w2):
    return pl.pallas_call(
        _combine_body,
        grid=(IROWS // CB,),
        in_specs=[
            pl.BlockSpec((CB, 128), lambda i: (i, 0)),
            pl.BlockSpec((CB, 128), lambda i: (i, 0)),
        ],
        out_specs=pl.BlockSpec((CB, 128), lambda i: (i, 0)),
        out_shape=jax.ShapeDtypeStruct((IROWS, 128), jnp.int32),
    )(t2, w2)


_mesh = plsc.VectorSubcoreMesh(core_axis_name="c", subcore_axis_name="s")


def _make_emb(p):
    @functools.partial(
        pl.kernel,
        mesh=_mesh,
        out_type=jax.ShapeDtypeStruct((PB * 208, D), jnp.float32),
        compiler_params=pltpu.CompilerParams(use_tc_tiling_on_sc=False),
        scratch_types=[
            pltpu.VMEM((K,), jnp.int32),
            pltpu.VMEM((K,), jnp.int32),
            pltpu.VMEM((K, D), jnp.float32),
            pltpu.VMEM((K, D), jnp.float32),
            pltpu.SemaphoreType.DMA,
            pltpu.SemaphoreType.DMA,
            pltpu.SemaphoreType.DMA,
            pltpu.SemaphoreType.DMA,
            pltpu.SemaphoreType.DMA,
            pltpu.SemaphoreType.DMA,
        ],
    )
    def _emb(cix_hbm, waug_hbm, out_hbm,
             cix0, cix1, rows0, rows1, i0, i1, g0, g1, o0, o1):
        wid = lax.axis_index("s") * NC + lax.axis_index("c")
        gbase = p * NP + wid * NPW
        bbase = wid * (NPW // L)

        CIX = (cix0, cix1)
        ROWS = (rows0, rows1)
        I = (i0, i1)
        G = (g0, g1)
        O = (o0, o1)

        def idx_copy(j, b):
            return pltpu.make_async_copy(
                cix_hbm.at[pl.ds(gbase + j * K, K)], CIX[b], I[b])

        def gathers(b):
            cs = []
            off = 0
            for sub in SUBS:
                cs.append(pltpu.make_async_copy(
                    waug_hbm.at[CIX[b].at[pl.ds(off, sub)]],
                    ROWS[b].at[pl.ds(off, sub)],
                    G[b],
                ))
                off += sub
            return cs

        def out_copies(j, b):
            return [
                pltpu.make_async_copy(
                    ROWS[b].at[pl.ds(i * L, L)],
                    out_hbm.at[pl.ds((bbase + j * 4 + i) * 208, L)],
                    O[b],
                )
                for i in range(4)
            ]

        idx_copy(0, 0).start()
        idx_copy(1, 1).start()
        for b in range(2):
            idx_copy(b, b).wait()
            for c in gathers(b):
                c.start()

        def body(jj, carry):
            for bb in range(2):
                j = jj * 2 + bb
                for c in gathers(bb):
                    c.wait()
                for c in out_copies(j, bb):
                    c.start()

                @pl.when(jj < NCHUNK // 2 - 1)
                def _prefetch(j=j, bb=bb):
                    idx_copy(j + 2, bb).start()
                    for c in out_copies(j, bb):
                        c.wait()
                    idx_copy(j + 2, bb).wait()
                    for c in gathers(bb):
                        c.start()
            return carry

        lax.fori_loop(0, NCHUNK // 2, body, 0)
        for c in out_copies(NCHUNK - 2, 0):
            c.wait()
        for c in out_copies(NCHUNK - 1, 1):
            c.wait()

    return _emb


_EMBS = [_make_emb(p) for p in range(P)]

BB3 = 16
SLABS = BB3 // 2
HL = L // 2


def _repack_body(in_ref, out_ref):
    x = in_ref[...]
    out_ref[:, pl.ds(0, HL, 2), :] = x[:, 0:HL, 0:D]
    out_ref[:, pl.ds(1, HL, 2), :] = x[:, 0:HL, D:]


def _repack_body_alias(prev_ref, in_ref, out_ref):
    del prev_ref
    _repack_body(in_ref, out_ref)


def _repack(out_prev, inter3, p):
    nblk = PB // BB3
    in_spec = pl.BlockSpec((BB3, 104, 128), lambda i: (i, 0, 0))
    out_spec = pl.BlockSpec((BB3, L, D), lambda i, p=p: (p * nblk + i, 0, 0))
    out_shape = jax.ShapeDtypeStruct((B, L, D), jnp.float32)
    if out_prev is None:
        return pl.pallas_call(
            _repack_body,
            grid=(nblk,),
            in_specs=[in_spec],
            out_specs=out_spec,
            out_shape=out_shape,
        )(inter3)
    return pl.pallas_call(
        _repack_body_alias,
        grid=(nblk,),
        in_specs=[pl.BlockSpec(memory_space=pl.ANY), in_spec],
        out_specs=out_spec,
        out_shape=out_shape,
        input_output_aliases={0: 0},
    )(out_prev, inter3)


def kernel(time, weekday, W_time, W_weekday):
    w_time2 = W_time.reshape(TROWS, 128)
    w_wd2 = jnp.concatenate([W_weekday, W_weekday], axis=1)
    w_aug = _augment(w_time2, w_wd2).reshape(NUM_WD * NUM_TIMES, D)
    t2 = time.reshape(IROWS, 128)
    w2 = weekday.reshape(IROWS, 128)
    cix = _combine(t2, w2).reshape(N)
    out = None
    for p in range(P):
        inter = _EMBS[p](cix, w_aug)
        inter3 = inter.reshape(PB, 104, 128)
        out = _repack(out, inter3, p)
    return out


# speedup vs baseline: 1.1709x; 1.1709x over previous
import functools

import jax
import jax.numpy as jnp
from jax import lax
from jax.experimental import pallas as pl
from jax.experimental.pallas import tpu as pltpu
from jax.experimental.pallas import tpu_sc as plsc

B, L, D = 16384, 200, 64
N = B * L
NUM_TIMES = 100000
NUM_WD = 7
NC, NS = 2, 16
NW = NC * NS

P = 8
PB = B // P
NP = PB * L
NPW = NP // NW
K = 800
SUBS = (128, 128, 128, 128, 128, 128, 32)
NCHUNK = NPW // K

TB = 2000
TROWS = NUM_TIMES * D // 128


def _augment_body(wt_ref, wwd_ref, out_ref):
    w = pl.program_id(0)
    out_ref[0, :, :] = wt_ref[...] + wwd_ref[w, :][None, :]


def _augment(w_time2, w_wd2):
    return pl.pallas_call(
        _augment_body,
        grid=(NUM_WD, TROWS // TB),
        in_specs=[
            pl.BlockSpec((TB, 128), lambda w, i: (i, 0)),
            pl.BlockSpec((NUM_WD, 128), lambda w, i: (0, 0)),
        ],
        out_specs=pl.BlockSpec((1, TB, 128), lambda w, i: (w, i, 0)),
        out_shape=jax.ShapeDtypeStruct((NUM_WD, TROWS, 128), jnp.float32),
    )(w_time2, w_wd2)


CB = 1600
IROWS = N // 128


def _combine_body(t_ref, w_ref, out_ref):
    out_ref[...] = w_ref[...] * NUM_TIMES + t_ref[...]


def _combine(t2, w2):
    return pl.pallas_call(
        _combine_body,
        grid=(IROWS // CB,),
        in_specs=[
            pl.BlockSpec((CB, 128), lambda i: (i, 0)),
            pl.BlockSpec((CB, 128), lambda i: (i, 0)),
        ],
        out_specs=pl.BlockSpec((CB, 128), lambda i: (i, 0)),
        out_shape=jax.ShapeDtypeStruct((IROWS, 128), jnp.int32),
    )(t2, w2)


_mesh = plsc.VectorSubcoreMesh(core_axis_name="c", subcore_axis_name="s")


def _make_emb(p):
    @functools.partial(
        pl.kernel,
        mesh=_mesh,
        out_type=jax.ShapeDtypeStruct((NP, D), jnp.float32),
        compiler_params=pltpu.CompilerParams(use_tc_tiling_on_sc=False),
        scratch_types=[
            pltpu.VMEM((K,), jnp.int32),
            pltpu.VMEM((K,), jnp.int32),
            pltpu.VMEM((K, D), jnp.float32),
            pltpu.VMEM((K, D), jnp.float32),
            pltpu.SemaphoreType.DMA,
            pltpu.SemaphoreType.DMA,
            pltpu.SemaphoreType.DMA,
            pltpu.SemaphoreType.DMA,
            pltpu.SemaphoreType.DMA,
            pltpu.SemaphoreType.DMA,
        ],
    )
    def _emb(cix_hbm, waug_hbm, out_hbm,
             cix0, cix1, rows0, rows1, i0, i1, g0, g1, o0, o1):
        wid = lax.axis_index("s") * NC + lax.axis_index("c")
        gbase = p * NP + wid * NPW
        obase = wid * NPW

        CIX = (cix0, cix1)
        ROWS = (rows0, rows1)
        I = (i0, i1)
        G = (g0, g1)
        O = (o0, o1)

        def idx_copy(j, b):
            return pltpu.make_async_copy(
                cix_hbm.at[pl.ds(gbase + j * K, K)], CIX[b], I[b])

        def gathers(b):
            cs = []
            off = 0
            for sub in SUBS:
                cs.append(pltpu.make_async_copy(
                    waug_hbm.at[CIX[b].at[pl.ds(off, sub)]],
                    ROWS[b].at[pl.ds(off, sub)],
                    G[b],
                ))
                off += sub
            return cs

        def out_copies(j, b):
            return [pltpu.make_async_copy(
                ROWS[b], out_hbm.at[pl.ds(obase + j * K, K)], O[b])]

        idx_copy(0, 0).start()
        idx_copy(1, 1).start()
        for b in range(2):
            idx_copy(b, b).wait()
            for c in gathers(b):
                c.start()

        def body(jj, carry):
            for bb in range(2):
                j = jj * 2 + bb
                for c in gathers(bb):
                    c.wait()
                for c in out_copies(j, bb):
                    c.start()

                @pl.when(jj < NCHUNK // 2 - 1)
                def _prefetch(j=j, bb=bb):
                    idx_copy(j + 2, bb).start()
                    for c in out_copies(j, bb):
                        c.wait()
                    idx_copy(j + 2, bb).wait()
                    for c in gathers(bb):
                        c.start()
            return carry

        lax.fori_loop(0, NCHUNK // 2, body, 0)
        for c in out_copies(NCHUNK - 2, 0):
            c.wait()
        for c in out_copies(NCHUNK - 1, 1):
            c.wait()

    return _emb


_EMBS = [_make_emb(p) for p in range(P)]

BB2 = 3200
HALF = BB2 // 2


def _repack_body(in_ref, out_ref):
    x = in_ref[...]
    out_ref[pl.ds(0, HALF, 2), :] = x[:, :D]
    out_ref[pl.ds(1, HALF, 2), :] = x[:, D:]


def _repack_body_alias(prev_ref, in_ref, out_ref):
    del prev_ref
    _repack_body(in_ref, out_ref)


def _repack(out_prev, inter2, p):
    nblk = NP // BB2
    in_spec = pl.BlockSpec((HALF, 128), lambda i: (i, 0))
    out_spec = pl.BlockSpec((BB2, D), lambda i, p=p: (p * nblk + i, 0))
    out_shape = jax.ShapeDtypeStruct((N, D), jnp.float32)
    if out_prev is None:
        return pl.pallas_call(
            _repack_body,
            grid=(nblk,),
            in_specs=[in_spec],
            out_specs=out_spec,
            out_shape=out_shape,
        )(inter2)
    return pl.pallas_call(
        _repack_body_alias,
        grid=(nblk,),
        in_specs=[pl.BlockSpec(memory_space=pl.ANY), in_spec],
        out_specs=out_spec,
        out_shape=out_shape,
        input_output_aliases={0: 0},
    )(out_prev, inter2)


def kernel(time, weekday, W_time, W_weekday):
    w_time2 = W_time.reshape(TROWS, 128)
    w_wd2 = jnp.concatenate([W_weekday, W_weekday], axis=1)
    w_aug = _augment(w_time2, w_wd2).reshape(NUM_WD * NUM_TIMES, D)
    t2 = time.reshape(IROWS, 128)
    w2 = weekday.reshape(IROWS, 128)
    cix = _combine(t2, w2).reshape(N)
    out = None
    for p in range(P):
        inter = _EMBS[p](cix, w_aug)
        inter2 = inter.reshape(NP * D // 128, 128)
        out = _repack(out, inter2, p)
    return out.reshape(B, L, D)


# speedup vs baseline: 1.2991x; 1.1095x over previous
import functools

import jax
import jax.numpy as jnp
from jax import lax
from jax.experimental import pallas as pl
from jax.experimental.pallas import tpu as pltpu
from jax.experimental.pallas import tpu_sc as plsc

B, L, D = 16384, 200, 64
N = B * L
NUM_TIMES = 100000
NUM_WD = 7
NC, NS = 2, 16
NW = NC * NS

P = 8
PB = B // P
NP = PB * L
NPW = NP // NW
K = 800
SUBS = (128, 128, 128, 128, 128, 128, 32)
NCHUNK = NPW // K

TB = 5000
TROWS = NUM_TIMES * D // 128


def _augment_body(wt_ref, wwd_ref, out_ref):
    w = pl.program_id(0)
    out_ref[0, :, :] = wt_ref[...] + wwd_ref[w, :][None, :]


def _augment(w_time2, w_wd2):
    return pl.pallas_call(
        _augment_body,
        grid=(NUM_WD, TROWS // TB),
        in_specs=[
            pl.BlockSpec((TB, 128), lambda w, i: (i, 0)),
            pl.BlockSpec((NUM_WD, 128), lambda w, i: (0, 0)),
        ],
        out_specs=pl.BlockSpec((1, TB, 128), lambda w, i: (w, i, 0)),
        out_shape=jax.ShapeDtypeStruct((NUM_WD, TROWS, 128), jnp.float32),
    )(w_time2, w_wd2)


CB = 1600
IROWS = N // 128


def _combine_body(t_ref, w_ref, out_ref):
    out_ref[...] = w_ref[...] * NUM_TIMES + t_ref[...]


def _combine(t2, w2):
    return pl.pallas_call(
        _combine_body,
        grid=(IROWS // CB,),
        in_specs=[
            pl.BlockSpec((CB, 128), lambda i: (i, 0)),
            pl.BlockSpec((CB, 128), lambda i: (i, 0)),
        ],
        out_specs=pl.BlockSpec((CB, 128), lambda i: (i, 0)),
        out_shape=jax.ShapeDtypeStruct((IROWS, 128), jnp.int32),
    )(t2, w2)


_mesh = plsc.VectorSubcoreMesh(core_axis_name="c", subcore_axis_name="s")


def _make_emb(p):
    @functools.partial(
        pl.kernel,
        mesh=_mesh,
        out_type=jax.ShapeDtypeStruct((NP, D), jnp.float32),
        compiler_params=pltpu.CompilerParams(use_tc_tiling_on_sc=False),
        scratch_types=[
            pltpu.VMEM((K,), jnp.int32),
            pltpu.VMEM((K,), jnp.int32),
            pltpu.VMEM((K, D), jnp.float32),
            pltpu.VMEM((K, D), jnp.float32),
            pltpu.SemaphoreType.DMA,
            pltpu.SemaphoreType.DMA,
            pltpu.SemaphoreType.DMA,
            pltpu.SemaphoreType.DMA,
            pltpu.SemaphoreType.DMA,
            pltpu.SemaphoreType.DMA,
        ],
    )
    def _emb(cix_hbm, waug_hbm, out_hbm,
             cix0, cix1, rows0, rows1, i0, i1, g0, g1, o0, o1):
        wid = lax.axis_index("s") * NC + lax.axis_index("c")
        gbase = p * NP + wid * NPW
        obase = wid * NPW

        CIX = (cix0, cix1)
        ROWS = (rows0, rows1)
        I = (i0, i1)
        G = (g0, g1)
        O = (o0, o1)

        def idx_copy(j, b):
            return pltpu.make_async_copy(
                cix_hbm.at[pl.ds(gbase + j * K, K)], CIX[b], I[b])

        def gathers(b):
            cs = []
            off = 0
            for sub in SUBS:
                cs.append(pltpu.make_async_copy(
                    waug_hbm.at[CIX[b].at[pl.ds(off, sub)]],
                    ROWS[b].at[pl.ds(off, sub)],
                    G[b],
                ))
                off += sub
            return cs

        def out_copies(j, b):
            return [pltpu.make_async_copy(
                ROWS[b], out_hbm.at[pl.ds(obase + j * K, K)], O[b])]

        idx_copy(0, 0).start()
        idx_copy(1, 1).start()
        for b in range(2):
            idx_copy(b, b).wait()
            for c in gathers(b):
                c.start()

        def body(jj, carry):
            for bb in range(2):
                j = jj * 2 + bb
                for c in gathers(bb):
                    c.wait()
                for c in out_copies(j, bb):
                    c.start()

                @pl.when(jj < NCHUNK // 2 - 1)
                def _prefetch(j=j, bb=bb):
                    idx_copy(j + 2, bb).start()
                    for c in out_copies(j, bb):
                        c.wait()
                    idx_copy(j + 2, bb).wait()
                    for c in gathers(bb):
                        c.start()
            return carry

        lax.fori_loop(0, NCHUNK // 2, body, 0)
        for c in out_copies(NCHUNK - 2, 0):
            c.wait()
        for c in out_copies(NCHUNK - 1, 1):
            c.wait()

    return _emb


_EMBS = [_make_emb(p) for p in range(P)]

BB2 = 6400
HALF = BB2 // 2


def _repack_body(in_ref, out_ref):
    x = in_ref[...]
    out_ref[pl.ds(0, HALF, 2), :] = x[:, :D]
    out_ref[pl.ds(1, HALF, 2), :] = x[:, D:]


def _repack_body_alias(prev_ref, in_ref, out_ref):
    del prev_ref
    _repack_body(in_ref, out_ref)


def _repack(out_prev, inter2, p):
    nblk = NP // BB2
    in_spec = pl.BlockSpec((HALF, 128), lambda i: (i, 0))
    out_spec = pl.BlockSpec((BB2, D), lambda i, p=p: (p * nblk + i, 0))
    out_shape = jax.ShapeDtypeStruct((N, D), jnp.float32)
    if out_prev is None:
        return pl.pallas_call(
            _repack_body,
            grid=(nblk,),
            in_specs=[in_spec],
            out_specs=out_spec,
            out_shape=out_shape,
        )(inter2)
    return pl.pallas_call(
        _repack_body_alias,
        grid=(nblk,),
        in_specs=[pl.BlockSpec(memory_space=pl.ANY), in_spec],
        out_specs=out_spec,
        out_shape=out_shape,
        input_output_aliases={0: 0},
    )(out_prev, inter2)


def kernel(time, weekday, W_time, W_weekday):
    w_time2 = W_time.reshape(TROWS, 128)
    w_wd2 = jnp.concatenate([W_weekday, W_weekday], axis=1)
    w_aug = _augment(w_time2, w_wd2).reshape(NUM_WD * NUM_TIMES, D)
    t2 = time.reshape(IROWS, 128)
    w2 = weekday.reshape(IROWS, 128)
    cix = _combine(t2, w2).reshape(N)
    out = None
    for p in range(P):
        inter = _EMBS[p](cix, w_aug)
        inter2 = inter.reshape(NP * D // 128, 128)
        out = _repack(out, inter2, p)
    return out.reshape(B, L, D)


# speedup vs baseline: 1.3202x; 1.0163x over previous
import functools

import jax
import jax.numpy as jnp
from jax import lax
from jax.experimental import pallas as pl
from jax.experimental.pallas import tpu as pltpu
from jax.experimental.pallas import tpu_sc as plsc

B, L, D = 16384, 200, 64
N = B * L
NUM_TIMES = 100000
NUM_WD = 7
NC, NS = 2, 16
NW = NC * NS

P = 8
PB = B // P
NP = PB * L
NPW = NP // NW
K = 800
SUBS = (128, 128, 128, 128, 128, 128, 32)
NCHUNK = NPW // K

TB = 10000
TROWS = NUM_TIMES * D // 128


def _augment_body(wt_ref, wwd_ref, out_ref):
    w = pl.program_id(0)
    out_ref[0, :, :] = wt_ref[...] + wwd_ref[w, :][None, :]


def _augment(w_time2, w_wd2):
    return pl.pallas_call(
        _augment_body,
        grid=(NUM_WD, TROWS // TB),
        in_specs=[
            pl.BlockSpec((TB, 128), lambda w, i: (i, 0)),
            pl.BlockSpec((NUM_WD, 128), lambda w, i: (0, 0)),
        ],
        out_specs=pl.BlockSpec((1, TB, 128), lambda w, i: (w, i, 0)),
        out_shape=jax.ShapeDtypeStruct((NUM_WD, TROWS, 128), jnp.float32),
    )(w_time2, w_wd2)


CB = 1600
IROWS = N // 128


def _combine_body(t_ref, w_ref, out_ref):
    out_ref[...] = w_ref[...] * NUM_TIMES + t_ref[...]


def _combine(t2, w2):
    return pl.pallas_call(
        _combine_body,
        grid=(IROWS // CB,),
        in_specs=[
            pl.BlockSpec((CB, 128), lambda i: (i, 0)),
            pl.BlockSpec((CB, 128), lambda i: (i, 0)),
        ],
        out_specs=pl.BlockSpec((CB, 128), lambda i: (i, 0)),
        out_shape=jax.ShapeDtypeStruct((IROWS, 128), jnp.int32),
    )(t2, w2)


_mesh = plsc.VectorSubcoreMesh(core_axis_name="c", subcore_axis_name="s")


def _make_emb(p):
    @functools.partial(
        pl.kernel,
        mesh=_mesh,
        out_type=jax.ShapeDtypeStruct((NP, D), jnp.float32),
        compiler_params=pltpu.CompilerParams(use_tc_tiling_on_sc=False),
        scratch_types=[
            pltpu.VMEM((K,), jnp.int32),
            pltpu.VMEM((K,), jnp.int32),
            pltpu.VMEM((K, D), jnp.float32),
            pltpu.VMEM((K, D), jnp.float32),
            pltpu.SemaphoreType.DMA,
            pltpu.SemaphoreType.DMA,
            pltpu.SemaphoreType.DMA,
            pltpu.SemaphoreType.DMA,
            pltpu.SemaphoreType.DMA,
            pltpu.SemaphoreType.DMA,
        ],
    )
    def _emb(cix_hbm, waug_hbm, out_hbm,
             cix0, cix1, rows0, rows1, i0, i1, g0, g1, o0, o1):
        wid = lax.axis_index("s") * NC + lax.axis_index("c")
        gbase = p * NP + wid * NPW
        obase = wid * NPW

        CIX = (cix0, cix1)
        ROWS = (rows0, rows1)
        I = (i0, i1)
        G = (g0, g1)
        O = (o0, o1)

        def idx_copy(j, b):
            return pltpu.make_async_copy(
                cix_hbm.at[pl.ds(gbase + j * K, K)], CIX[b], I[b])

        def gathers(b):
            cs = []
            off = 0
            for sub in SUBS:
                cs.append(pltpu.make_async_copy(
                    waug_hbm.at[CIX[b].at[pl.ds(off, sub)]],
                    ROWS[b].at[pl.ds(off, sub)],
                    G[b],
                ))
                off += sub
            return cs

        def out_copies(j, b):
            return [pltpu.make_async_copy(
                ROWS[b], out_hbm.at[pl.ds(obase + j * K, K)], O[b])]

        idx_copy(0, 0).start()
        idx_copy(1, 1).start()
        for b in range(2):
            idx_copy(b, b).wait()
            for c in gathers(b):
                c.start()

        def body(jj, carry):
            for bb in range(2):
                j = jj * 2 + bb
                for c in gathers(bb):
                    c.wait()
                for c in out_copies(j, bb):
                    c.start()

                @pl.when(jj < NCHUNK // 2 - 1)
                def _prefetch(j=j, bb=bb):
                    idx_copy(j + 2, bb).start()
                    for c in out_copies(j, bb):
                        c.wait()
                    idx_copy(j + 2, bb).wait()
                    for c in gathers(bb):
                        c.start()
            return carry

        lax.fori_loop(0, NCHUNK // 2, body, 0)
        for c in out_copies(NCHUNK - 2, 0):
            c.wait()
        for c in out_copies(NCHUNK - 1, 1):
            c.wait()

    return _emb


_EMBS = [_make_emb(p) for p in range(P)]

BB2 = 12800
HALF = BB2 // 2


def _repack_body(in_ref, out_ref):
    x = in_ref[...]
    out_ref[pl.ds(0, HALF, 2), :] = x[:, :D]
    out_ref[pl.ds(1, HALF, 2), :] = x[:, D:]


def _repack_body_alias(prev_ref, in_ref, out_ref):
    del prev_ref
    _repack_body(in_ref, out_ref)


def _repack(out_prev, inter2, p):
    nblk = NP // BB2
    in_spec = pl.BlockSpec((HALF, 128), lambda i: (i, 0))
    out_spec = pl.BlockSpec((BB2, D), lambda i, p=p: (p * nblk + i, 0))
    out_shape = jax.ShapeDtypeStruct((N, D), jnp.float32)
    if out_prev is None:
        return pl.pallas_call(
            _repack_body,
            grid=(nblk,),
            in_specs=[in_spec],
            out_specs=out_spec,
            out_shape=out_shape,
        )(inter2)
    return pl.pallas_call(
        _repack_body_alias,
        grid=(nblk,),
        in_specs=[pl.BlockSpec(memory_space=pl.ANY), in_spec],
        out_specs=out_spec,
        out_shape=out_shape,
        input_output_aliases={0: 0},
    )(out_prev, inter2)


def kernel(time, weekday, W_time, W_weekday):
    w_time2 = W_time.reshape(TROWS, 128)
    w_wd2 = jnp.concatenate([W_weekday, W_weekday], axis=1)
    w_aug = _augment(w_time2, w_wd2).reshape(NUM_WD * NUM_TIMES, D)
    t2 = time.reshape(IROWS, 128)
    w2 = weekday.reshape(IROWS, 128)
    cix = _combine(t2, w2).reshape(N)
    out = None
    for p in range(P):
        inter = _EMBS[p](cix, w_aug)
        inter2 = inter.reshape(NP * D // 128, 128)
        out = _repack(out, inter2, p)
    return out.reshape(B, L, D)
